# baseline (speedup 1.0000x reference)
import functools

import jax
import jax.numpy as jnp
import numpy as np
from jax import lax
from jax.experimental import pallas as pl
from jax.experimental.pallas import tpu as pltpu
from jax.experimental.pallas import tpu_sc as plsc

_VOCAB = 1000000
_D = 64
_DP = 128
_B = 4096
_L = 200

_NC = 2
_NS = 16
_NW = _NC * _NS
_ITEMS = _B // _NW
_PAIRS = _ITEMS // 2

_CVT_BLK = 8000

_HI_MASK = np.int32(-65536)
_SHIFT = np.int32(16)


def _cvt_body(w_ref, out_ref):
    wb = w_ref[...].astype(jnp.bfloat16)
    pad = jnp.zeros((_CVT_BLK, _DP - _D), jnp.bfloat16)
    out_ref[...] = jnp.concatenate([wb, pad], axis=1)


def _convert_table(weights):
    return pl.pallas_call(
        _cvt_body,
        grid=(_VOCAB // _CVT_BLK,),
        in_specs=[pl.BlockSpec((_CVT_BLK, _D), lambda i: (i, 0))],
        out_specs=pl.BlockSpec((_CVT_BLK, _DP), lambda i: (i, 0)),
        out_shape=jax.ShapeDtypeStruct((_VOCAB, _DP), jnp.bfloat16),
        compiler_params=pltpu.CompilerParams(
            dimension_semantics=("arbitrary",),
        ),
    )(weights)


def _accum_item(buf, acc):
    def body(k, carry):
        e0, o0, e1, o1 = carry
        w0 = plsc.bitcast(buf[k, pl.ds(0, 32)], jnp.int32)
        w1 = plsc.bitcast(buf[k, pl.ds(32, 32)], jnp.int32)
        e0 = e0 + plsc.bitcast(lax.shift_left(w0, _SHIFT), jnp.float32)
        o0 = o0 + plsc.bitcast(lax.bitwise_and(w0, _HI_MASK), jnp.float32)
        e1 = e1 + plsc.bitcast(lax.shift_left(w1, _SHIFT), jnp.float32)
        o1 = o1 + plsc.bitcast(lax.bitwise_and(w1, _HI_MASK), jnp.float32)
        return e0, o0, e1, o1

    return lax.fori_loop(0, _L, body, acc, unroll=8)


def _store_row(out_v, i, acc, inv_l, col2):
    e0, o0, e1, o1 = acc
    row = jnp.full((16,), i, jnp.int32)
    plsc.store_scatter(out_v, [row, col2], e0 * inv_l)
    plsc.store_scatter(out_v, [row, col2 + 1], o0 * inv_l)
    plsc.store_scatter(out_v, [row, col2 + 32], e1 * inv_l)
    plsc.store_scatter(out_v, [row, col2 + 33], o1 * inv_l)


def _pooled_embedding(x, wb):
    mesh = plsc.VectorSubcoreMesh(core_axis_name="c", subcore_axis_name="s")

    @functools.partial(
        pl.kernel,
        mesh=mesh,
        out_type=jax.ShapeDtypeStruct((_B, _D), jnp.float32),
        compiler_params=pltpu.CompilerParams(
            use_tc_tiling_on_sc=False, needs_layout_passes=False
        ),
        scratch_types=[
            pltpu.VMEM((_ITEMS, _L), jnp.int32),
            pltpu.VMEM((_L, _DP), jnp.bfloat16),
            pltpu.VMEM((_L, _DP), jnp.bfloat16),
            pltpu.VMEM((_ITEMS, _D), jnp.float32),
            pltpu.SemaphoreType.DMA,
            pltpu.SemaphoreType.DMA,
        ],
    )
    def k(x_hbm, w_hbm, out_hbm, idx_v, buf_a, buf_b, out_v, sem_a, sem_b):
        wid = lax.axis_index("s") * _NC + lax.axis_index("c")
        pltpu.sync_copy(x_hbm.at[pl.ds(wid * _ITEMS, _ITEMS)], idx_v)

        pltpu.async_copy(w_hbm.at[idx_v.at[0]], buf_a, sem_a)
        pltpu.async_copy(w_hbm.at[idx_v.at[1]], buf_b, sem_b)

        inv_l = jnp.float32(1.0 / _L)
        zero = jnp.zeros((16,), jnp.float32)
        col2 = jnp.arange(16, dtype=jnp.int32) * 2

        def pair(p, _):
            i = 2 * p
            pltpu.make_async_copy(w_hbm.at[idx_v.at[0]], buf_a, sem_a).wait()
            acc = _accum_item(buf_a, (zero, zero, zero, zero))

            @pl.when(p < _PAIRS - 1)
            def _():
                pltpu.async_copy(w_hbm.at[idx_v.at[i + 2]], buf_a, sem_a)

            _store_row(out_v, i, acc, inv_l, col2)

            pltpu.make_async_copy(w_hbm.at[idx_v.at[1]], buf_b, sem_b).wait()
            acc = _accum_item(buf_b, (zero, zero, zero, zero))

            @pl.when(p < _PAIRS - 1)
            def _():
                pltpu.async_copy(w_hbm.at[idx_v.at[i + 3]], buf_b, sem_b)

            _store_row(out_v, i + 1, acc, inv_l, col2)
            return 0

        lax.fori_loop(0, _PAIRS, pair, 0)

        pltpu.sync_copy(out_v, out_hbm.at[pl.ds(wid * _ITEMS, _ITEMS)])

    return k(x, wb)


def kernel(x, weights):
    wb = _convert_table(weights)
    return _pooled_embedding(x.astype(jnp.int32), wb)


# speedup vs baseline: 1.6799x; 1.6799x over previous
import functools

import jax
import jax.numpy as jnp
from jax import lax
from jax.experimental import pallas as pl
from jax.experimental.pallas import tpu as pltpu
from jax.experimental.pallas import tpu_sc as plsc

_VOCAB = 1000000
_D = 64
_DP = 128
_B = 4096
_L = 200

_CHUNK = 100
_NC = 2
_NS = 16
_NW = _NC * _NS
_ITEMS = _B // _NW
_NCHUNK = _ITEMS * (_L // _CHUNK)


def _accum_chunk(buf, acc):
    def body(k, carry):
        c0, c1, c2, c3 = carry
        c0 = c0 + buf[k, pl.ds(0, 16)]
        c1 = c1 + buf[k, pl.ds(16, 16)]
        c2 = c2 + buf[k, pl.ds(32, 16)]
        c3 = c3 + buf[k, pl.ds(48, 16)]
        return c0, c1, c2, c3

    return lax.fori_loop(0, _CHUNK, body, acc, unroll=8)


def _pooled_embedding(x2, wd):
    mesh = plsc.VectorSubcoreMesh(core_axis_name="c", subcore_axis_name="s")

    @functools.partial(
        pl.kernel,
        mesh=mesh,
        out_type=jax.ShapeDtypeStruct((_B, _D), jnp.float32),
        compiler_params=pltpu.CompilerParams(use_tc_tiling_on_sc=True),
        scratch_types=[
            pltpu.VMEM((_NCHUNK, _CHUNK), jnp.int32),
            pltpu.VMEM((_CHUNK, _DP), jnp.float32),
            pltpu.VMEM((_CHUNK, _DP), jnp.float32),
            pltpu.VMEM((_ITEMS, _D), jnp.float32),
            pltpu.SemaphoreType.DMA,
            pltpu.SemaphoreType.DMA,
        ],
    )
    def k(x_hbm, w_hbm, out_hbm, idx_v, buf_a, buf_b, out_v, sem_a, sem_b):
        wid = lax.axis_index("s") * _NC + lax.axis_index("c")
        pltpu.sync_copy(x_hbm.at[pl.ds(wid * _NCHUNK, _NCHUNK)], idx_v)

        pltpu.async_copy(w_hbm.at[idx_v.at[0]], buf_a, sem_a)
        pltpu.async_copy(w_hbm.at[idx_v.at[1]], buf_b, sem_b)

        inv_l = jnp.float32(1.0 / _L)
        zero = jnp.zeros((16,), jnp.float32)

        def item(i, _):
            pltpu.make_async_copy(w_hbm.at[idx_v.at[0]], buf_a, sem_a).wait()
            a0, a1, a2, a3 = _accum_chunk(buf_a, (zero, zero, zero, zero))

            @pl.when(i < _ITEMS - 1)
            def _():
                pltpu.async_copy(w_hbm.at[idx_v.at[2 * i + 2]], buf_a, sem_a)

            pltpu.make_async_copy(w_hbm.at[idx_v.at[1]], buf_b, sem_b).wait()
            a0, a1, a2, a3 = _accum_chunk(buf_b, (a0, a1, a2, a3))

            @pl.when(i < _ITEMS - 1)
            def _():
                pltpu.async_copy(w_hbm.at[idx_v.at[2 * i + 3]], buf_b, sem_b)

            out_v[i, pl.ds(0, 16)] = a0 * inv_l
            out_v[i, pl.ds(16, 16)] = a1 * inv_l
            out_v[i, pl.ds(32, 16)] = a2 * inv_l
            out_v[i, pl.ds(48, 16)] = a3 * inv_l
            return 0

        lax.fori_loop(0, _ITEMS, item, 0)

        pltpu.sync_copy(out_v, out_hbm.at[pl.ds(wid * _ITEMS, _ITEMS)])

    return k(x2, wd)


def kernel(x, weights):
    wd = jnp.concatenate([weights, weights], axis=1)
    x2 = x.astype(jnp.int32).reshape(-1, _CHUNK)
    return _pooled_embedding(x2, wd)


# speedup vs baseline: 2.1314x; 1.2688x over previous
import functools

import jax
import jax.numpy as jnp
from jax import lax
from jax.experimental import pallas as pl
from jax.experimental.pallas import tpu as pltpu
from jax.experimental.pallas import tpu_sc as plsc

_VOCAB = 1000000
_D = 64
_B = 4096
_L = 200

_NC = 2
_NS = 16
_NW = _NC * _NS
_ITEMS = _B // _NW
_PAIRS = _ITEMS // 2


def _accum_item(buf, a0, a1, a2, a3):
    def body(k, carry):
        c0, c1, c2, c3 = carry
        c0 = c0 + buf[k, pl.ds(0, 16)]
        c1 = c1 + buf[k, pl.ds(16, 16)]
        c2 = c2 + buf[k, pl.ds(32, 16)]
        c3 = c3 + buf[k, pl.ds(48, 16)]
        return c0, c1, c2, c3

    return lax.fori_loop(0, _L, body, (a0, a1, a2, a3), unroll=8)


def _pooled_embedding(x, weights):
    mesh = plsc.VectorSubcoreMesh(core_axis_name="c", subcore_axis_name="s")

    @functools.partial(
        pl.kernel,
        mesh=mesh,
        out_type=jax.ShapeDtypeStruct((_B, _D), jnp.float32),
        compiler_params=pltpu.CompilerParams(use_tc_tiling_on_sc=False),
        scratch_types=[
            pltpu.VMEM((_ITEMS, _L), jnp.int32),
            pltpu.VMEM((_L, _D), jnp.float32),
            pltpu.VMEM((_L, _D), jnp.float32),
            pltpu.VMEM((_ITEMS, _D), jnp.float32),
            pltpu.SemaphoreType.DMA,
            pltpu.SemaphoreType.DMA,
        ],
    )
    def k(x_hbm, w_hbm, out_hbm, idx_v, buf_a, buf_b, out_v, sem_a, sem_b):
        wid = lax.axis_index("s") * _NC + lax.axis_index("c")
        pltpu.sync_copy(x_hbm.at[pl.ds(wid * _ITEMS, _ITEMS)], idx_v)

        pltpu.async_copy(w_hbm.at[idx_v.at[0]], buf_a, sem_a)
        pltpu.async_copy(w_hbm.at[idx_v.at[1]], buf_b, sem_b)

        inv_l = jnp.float32(1.0 / _L)
        zero = jnp.zeros((16,), jnp.float32)

        def pair(p, _):
            i = 2 * p
            pltpu.make_async_copy(w_hbm.at[idx_v.at[0]], buf_a, sem_a).wait()
            a0, a1, a2, a3 = _accum_item(buf_a, zero, zero, zero, zero)

            @pl.when(p < _PAIRS - 1)
            def _():
                pltpu.async_copy(w_hbm.at[idx_v.at[i + 2]], buf_a, sem_a)

            out_v[i, pl.ds(0, 16)] = a0 * inv_l
            out_v[i, pl.ds(16, 16)] = a1 * inv_l
            out_v[i, pl.ds(32, 16)] = a2 * inv_l
            out_v[i, pl.ds(48, 16)] = a3 * inv_l

            pltpu.make_async_copy(w_hbm.at[idx_v.at[1]], buf_b, sem_b).wait()
            b0, b1, b2, b3 = _accum_item(buf_b, zero, zero, zero, zero)

            @pl.when(p < _PAIRS - 1)
            def _():
                pltpu.async_copy(w_hbm.at[idx_v.at[i + 3]], buf_b, sem_b)

            out_v[i + 1, pl.ds(0, 16)] = b0 * inv_l
            out_v[i + 1, pl.ds(16, 16)] = b1 * inv_l
            out_v[i + 1, pl.ds(32, 16)] = b2 * inv_l
            out_v[i + 1, pl.ds(48, 16)] = b3 * inv_l
            return 0

        lax.fori_loop(0, _PAIRS, pair, 0)

        pltpu.sync_copy(out_v, out_hbm.at[pl.ds(wid * _ITEMS, _ITEMS)])

    return k(x, weights)


def kernel(x, weights):
    return _pooled_embedding(x.astype(jnp.int32), weights)
